# baseline (speedup 1.0000x reference)
import functools

import jax
import jax.numpy as jnp
from jax import lax
from jax.experimental import pallas as pl
from jax.experimental.pallas import tpu as pltpu
from jax.experimental.pallas import tpu_sc as plsc

N = 800000
D = 64
NUM_SEG = 50000

NC = 2
NS = 16
NW = NC * NS

S_PAD = 1568
T1 = 256
BLK = 8000


def _fx_body(x_ref, w_ref, b_ref, g_ref, be_ref, o_ref):
    x = x_ref[...]
    w = w_ref[...]
    wc = w - jnp.mean(w, axis=0, keepdims=True)
    z = jnp.zeros((D, D), jnp.float32)
    wc2 = jnp.concatenate(
        [jnp.concatenate([wc, z], axis=1),
         jnp.concatenate([z, wc], axis=1)], axis=0)
    bc = b_ref[...] - jnp.mean(b_ref[...])
    bc2 = jnp.concatenate([bc, bc], axis=1)
    d = lax.dot_general(x, wc2, (((1,), (1,)), ((), ())),
                        preferred_element_type=jnp.float32) + bc2
    r = jax.lax.broadcasted_iota(jnp.int32, (2 * D, 2 * D), 0) // D
    cc = jax.lax.broadcasted_iota(jnp.int32, (2 * D, 2 * D), 1) // D
    ones2 = jnp.where(r == cc, 1.0 / D, 0.0)
    varb = lax.dot_general(d * d, ones2, (((1,), (0,)), ((), ())),
                           preferred_element_type=jnp.float32)
    g2 = jnp.concatenate([g_ref[...], g_ref[...]], axis=1)
    be2 = jnp.concatenate([be_ref[...], be_ref[...]], axis=1)
    hn = d * lax.rsqrt(varb + 1e-5) * g2 + be2
    o_ref[...] = jnp.maximum(hn, 0.0)


def _fx_tc(X2, W, b2, g2, be2):
    return pl.pallas_call(
        _fx_body,
        grid=(N // BLK,),
        in_specs=[
            pl.BlockSpec((BLK // 2, 2 * D), lambda i: (i, 0)),
            pl.BlockSpec((D, D), lambda i: (0, 0)),
            pl.BlockSpec((1, D), lambda i: (0, 0)),
            pl.BlockSpec((1, D), lambda i: (0, 0)),
            pl.BlockSpec((1, D), lambda i: (0, 0)),
        ],
        out_specs=pl.BlockSpec((BLK // 2, 2 * D), lambda i: (i, 0)),
        out_shape=jax.ShapeDtypeStruct(((N + BLK) // 2, 2 * D), jnp.float32),
    )(X2, W, b2, g2, be2)


def _segbc_body(fx_hbm, ids_hbm, bounds_hbm, res_hbm,
                stage_v, fx_v, ids_v, bounds_v, ld_sems, wr_sems):
    c = lax.axis_index("c")
    s = lax.axis_index("s")
    w = s * NC + c
    pltpu.sync_copy(bounds_hbm.at[w], bounds_v)
    bv = bounds_v[...]
    r0 = bv[0]
    r1 = bv[1]
    s0 = w * S_PAD
    tb0 = (r0 // T1) * T1
    ntiles = (r1 - tb0 + (T1 - 1)) // T1
    nsteps = 3 * ((ntiles + 2) // 3)

    def ld_start(t, b):
        base = tb0 + t * T1
        pltpu.async_copy(fx_hbm.at[pl.ds(base, T1)], fx_v.at[b], ld_sems.at[b])
        pltpu.async_copy(ids_hbm.at[pl.ds(base, T1)], ids_v.at[b],
                         ld_sems.at[b])

    def ld_wait(b):
        pltpu.make_async_copy(fx_hbm.at[pl.ds(0, T1)], fx_v.at[b],
                              ld_sems.at[b]).wait()
        pltpu.make_async_copy(ids_hbm.at[pl.ds(0, T1)], ids_v.at[b],
                              ld_sems.at[b]).wait()

    def wrl_wait(b):
        pltpu.make_async_copy(fx_v.at[b],
                              res_hbm.at[pl.ds(0, T1), pl.ds(0, D)],
                              wr_sems.at[b]).wait()

    @pl.when(nsteps > 0)
    def _():
        ld_start(0, 0)
        ld_start(1, 1)

    def pair1(k, carry):
        for b in range(3):
            t = 3 * k + b
            base = tb0 + t * T1
            ld_wait(b)

            @pl.when(t < ntiles)
            def _(b=b, base=base):
                pltpu.async_copy(fx_v.at[b],
                                 res_hbm.at[pl.ds(base, T1), pl.ds(0, D)],
                                 wr_sems.at[b])

            def grp1(g, c2):
                prev_id, a0, a1, a2, a3 = c2
                gb = g * 16
                idvec = ids_v[b, pl.ds(gb, 16)]
                for jj in range(16):
                    j = gb + jj
                    idx = base + j
                    idj = idvec[jj]
                    valid = jnp.logical_and(idx >= r0, idx < r1)
                    neq = idj != prev_id
                    slot = jnp.where(valid, idj - s0, S_PAD)
                    v0 = fx_v[b, j, pl.ds(0, 16)]
                    v1 = fx_v[b, j, pl.ds(16, 16)]
                    v2 = fx_v[b, j, pl.ds(32, 16)]
                    v3 = fx_v[b, j, pl.ds(48, 16)]
                    a0 = jnp.where(neq, v0, jnp.maximum(a0, v0))
                    a1 = jnp.where(neq, v1, jnp.maximum(a1, v1))
                    a2 = jnp.where(neq, v2, jnp.maximum(a2, v2))
                    a3 = jnp.where(neq, v3, jnp.maximum(a3, v3))
                    stage_v[slot, pl.ds(0, 32)] = plsc.pack(
                        a0, a1, format=plsc.PackFormat.INTERLEAVED)
                    stage_v[slot, pl.ds(32, 32)] = plsc.pack(
                        a2, a3, format=plsc.PackFormat.INTERLEAVED)
                    prev_id = idj
                return (prev_id, a0, a1, a2, a3)

            carry = lax.fori_loop(0, T1 // 16, grp1, carry)

            @pl.when(jnp.logical_and(t - 1 >= 0, t - 1 < ntiles))
            def _(b=b):
                wrl_wait((b + 2) % 3)

            @pl.when(t + 2 < nsteps)
            def _(t=t, b=b):
                ld_start(t + 2, (b + 2) % 3)

        return carry

    z = jnp.zeros((16,), jnp.float32)
    lax.fori_loop(0, nsteps // 3, pair1, (jnp.int32(-1), z, z, z, z))

    @pl.when(jnp.logical_and(nsteps > 0, nsteps - 1 < ntiles))
    def _():
        wrl_wait(2)

    def ld2_start(t, b):
        base = tb0 + t * T1
        pltpu.async_copy(ids_hbm.at[pl.ds(base, T1)], ids_v.at[b],
                         ld_sems.at[b])

    def ld2_wait(b):
        pltpu.make_async_copy(ids_hbm.at[pl.ds(0, T1)], ids_v.at[b],
                              ld_sems.at[b]).wait()

    def wrr_wait(b):
        pltpu.make_async_copy(fx_v.at[b],
                              res_hbm.at[pl.ds(0, T1), pl.ds(D, D)],
                              wr_sems.at[b]).wait()

    def jlo_of(t):
        return jnp.maximum(r0 - (tb0 + t * T1), 0)

    def jhi_of(t):
        return jnp.minimum(r1 - (tb0 + t * T1), T1)

    def full_of(t):
        return jnp.logical_and(jlo_of(t) == 0, jhi_of(t) == T1)

    @pl.when(nsteps > 0)
    def _():
        ld2_start(0, 0)
        ld2_start(1, 1)

    def pair2(k, _):
        for b in range(3):
            t = 3 * k + b
            base = tb0 + t * T1
            ld2_wait(b)

            @pl.when(jnp.logical_and(t - 3 >= 0, full_of(t - 3)))
            def _(b=b):
                wrr_wait(b)

            jlo = jlo_of(t)
            jhi = jhi_of(t)

            def grp2(g, _c):
                gb = g * 16
                idvec = ids_v[b, pl.ds(gb, 16)]
                for jj in range(16):
                    j = gb + jj
                    slot = jnp.clip(idvec[jj] - s0, 0, S_PAD)
                    p01 = stage_v[slot, pl.ds(0, 32)]
                    p23 = stage_v[slot, pl.ds(32, 32)]
                    u0, u1 = plsc.unpack(p01,
                                         format=plsc.PackFormat.INTERLEAVED)
                    u2, u3 = plsc.unpack(p23,
                                         format=plsc.PackFormat.INTERLEAVED)
                    fx_v[b, j, pl.ds(0, 16)] = u0
                    fx_v[b, j, pl.ds(16, 16)] = u1
                    fx_v[b, j, pl.ds(32, 16)] = u2
                    fx_v[b, j, pl.ds(48, 16)] = u3
                return 0

            lax.fori_loop(jlo // 16, (jhi + 15) // 16, grp2, 0)

            full = jnp.logical_and(jlo == 0, jhi == T1)

            @pl.when(full)
            def _(b=b, base=base):
                pltpu.async_copy(fx_v.at[b],
                                 res_hbm.at[pl.ds(base, T1), pl.ds(D, D)],
                                 wr_sems.at[b])

            @pl.when(jnp.logical_and(jnp.logical_not(full), jhi > jlo))
            def _(b=b, base=base, jlo=jlo, jhi=jhi):
                rem = jhi - jlo
                off = jlo
                for sz in (128, 64, 32, 16, 8, 4, 2, 1):
                    cond = (rem & sz) != 0

                    @pl.when(cond)
                    def _(b=b, off=off, sz=sz):
                        pltpu.sync_copy(
                            fx_v.at[b, pl.ds(off, sz)],
                            res_hbm.at[pl.ds(base + off, sz), pl.ds(D, D)])

                    off = off + jnp.where(cond, sz, 0)

            @pl.when(t + 2 < nsteps)
            def _(t=t, b=b):
                ld2_start(t + 2, (b + 2) % 3)

        return 0

    lax.fori_loop(0, nsteps // 3, pair2, 0)

    for bb in range(3):
        tt = nsteps - 3 + bb

        @pl.when(jnp.logical_and(tt >= 0, full_of(tt)))
        def _(bb=bb):
            wrr_wait(bb)


def _segbc_sc(fX, ids_pad, bounds_tbl):
    mesh = plsc.VectorSubcoreMesh(core_axis_name="c", subcore_axis_name="s")
    return pl.kernel(
        _segbc_body,
        out_type=jax.ShapeDtypeStruct((N, 2 * D), jnp.float32),
        mesh=mesh,
        compiler_params=pltpu.CompilerParams(use_tc_tiling_on_sc=False,
                                             needs_layout_passes=False),
        scratch_types=[
            pltpu.VMEM((S_PAD + 1, D), jnp.bfloat16),
            pltpu.VMEM((3, T1, D), jnp.float32),
            pltpu.VMEM((3, T1), jnp.int32),
            pltpu.VMEM((16,), jnp.int32),
            pltpu.SemaphoreType.DMA((3,)),
            pltpu.SemaphoreType.DMA((3,)),
        ],
    )(fX, ids_pad, bounds_tbl)


def kernel(X, W, b, gamma, beta, i):
    ids = i.astype(jnp.int32)
    X2 = jnp.concatenate([X[0::2], X[1::2]], axis=1)
    fX = _fx_tc(X2, W, b.reshape(1, D), gamma.reshape(1, D),
                beta.reshape(1, D)).reshape(N + BLK, D)
    seg_bounds = jnp.arange(NW + 1, dtype=jnp.int32) * S_PAD
    row_bounds = jnp.searchsorted(ids, seg_bounds, side="left").astype(jnp.int32)
    bounds_tbl = jnp.concatenate(
        [row_bounds[:-1, None], row_bounds[1:, None],
         jnp.zeros((NW, 14), jnp.int32)], axis=1)
    ids_pad = jnp.concatenate([ids, jnp.zeros((T1 + 8,), jnp.int32)])
    return _segbc_sc(fX, ids_pad, bounds_tbl)


# speedup vs baseline: 8.1102x; 8.1102x over previous
import functools

import jax
import jax.numpy as jnp
from jax import lax
from jax.experimental import pallas as pl
from jax.experimental.pallas import tpu as pltpu
from jax.experimental.pallas import tpu_sc as plsc

N = 800000
D = 64
NUM_SEG = 50000

NC = 2
NS = 16
NW = NC * NS

S_PAD = 1568
T1 = 256
BLK = 16000


def _fx_body(x_ref, w_ref, b_ref, g_ref, be_ref, o_ref):
    x = x_ref[...]
    w = w_ref[...]
    wc = w - jnp.mean(w, axis=0, keepdims=True)
    z = jnp.zeros((D, D), jnp.float32)
    wc2 = jnp.concatenate(
        [jnp.concatenate([wc, z], axis=1),
         jnp.concatenate([z, wc], axis=1)], axis=0)
    bc = b_ref[...] - jnp.mean(b_ref[...])
    bc2 = jnp.concatenate([bc, bc], axis=1)
    d = lax.dot_general(x, wc2, (((1,), (1,)), ((), ())),
                        preferred_element_type=jnp.float32) + bc2
    r = jax.lax.broadcasted_iota(jnp.int32, (2 * D, 2 * D), 0) // D
    cc = jax.lax.broadcasted_iota(jnp.int32, (2 * D, 2 * D), 1) // D
    ones2 = jnp.where(r == cc, 1.0 / D, 0.0)
    varb = lax.dot_general(d * d, ones2, (((1,), (0,)), ((), ())),
                           preferred_element_type=jnp.float32)
    g2 = jnp.concatenate([g_ref[...], g_ref[...]], axis=1)
    be2 = jnp.concatenate([be_ref[...], be_ref[...]], axis=1)
    hn = d * lax.rsqrt(varb + 1e-5) * g2 + be2
    o_ref[...] = jnp.maximum(hn, 0.0)


def _fx_tc(X2, W, b2, g2, be2):
    return pl.pallas_call(
        _fx_body,
        grid=(N // BLK,),
        in_specs=[
            pl.BlockSpec((BLK // 2, 2 * D), lambda i: (i, 0)),
            pl.BlockSpec((D, D), lambda i: (0, 0)),
            pl.BlockSpec((1, D), lambda i: (0, 0)),
            pl.BlockSpec((1, D), lambda i: (0, 0)),
            pl.BlockSpec((1, D), lambda i: (0, 0)),
        ],
        out_specs=pl.BlockSpec((BLK // 2, 2 * D), lambda i: (i, 0)),
        out_shape=jax.ShapeDtypeStruct(((N + BLK) // 2, 2 * D), jnp.float32),
    )(X2, W, b2, g2, be2)


def _segbc_body(fx_hbm, ids_hbm, bounds_hbm, res_hbm,
                stage_v, fx_v, ids_v, bounds_v, ld_sems, wr_sems):
    c = lax.axis_index("c")
    s = lax.axis_index("s")
    w = s * NC + c
    pltpu.sync_copy(bounds_hbm.at[w], bounds_v)
    bv = bounds_v[...]
    r0 = bv[0]
    r1 = bv[1]
    s0 = w * S_PAD
    tb0 = (r0 // T1) * T1
    ntiles = (r1 - tb0 + (T1 - 1)) // T1
    nsteps = 3 * ((ntiles + 2) // 3)

    def ld_start(t, b):
        base = tb0 + t * T1
        pltpu.async_copy(fx_hbm.at[pl.ds(base, T1)], fx_v.at[b], ld_sems.at[b])
        pltpu.async_copy(ids_hbm.at[pl.ds(base, T1)], ids_v.at[b],
                         ld_sems.at[b])

    def ld_wait(b):
        pltpu.make_async_copy(fx_hbm.at[pl.ds(0, T1)], fx_v.at[b],
                              ld_sems.at[b]).wait()
        pltpu.make_async_copy(ids_hbm.at[pl.ds(0, T1)], ids_v.at[b],
                              ld_sems.at[b]).wait()

    def wrl_wait(b):
        pltpu.make_async_copy(fx_v.at[b],
                              res_hbm.at[pl.ds(0, T1), pl.ds(0, D)],
                              wr_sems.at[b]).wait()

    @pl.when(nsteps > 0)
    def _():
        ld_start(0, 0)
        ld_start(1, 1)

    def pair1(k, carry):
        for b in range(3):
            t = 3 * k + b
            base = tb0 + t * T1
            ld_wait(b)

            @pl.when(t < ntiles)
            def _(b=b, base=base):
                pltpu.async_copy(fx_v.at[b],
                                 res_hbm.at[pl.ds(base, T1), pl.ds(0, D)],
                                 wr_sems.at[b])

            def grp1(g, c2):
                prev_id, a0, a1, a2, a3 = c2
                gb = g * 16
                idvec = ids_v[b, pl.ds(gb, 16)]
                for jj in range(16):
                    j = gb + jj
                    idx = base + j
                    idj = idvec[jj]
                    valid = jnp.logical_and(idx >= r0, idx < r1)
                    neq = idj != prev_id
                    slot = jnp.where(valid, idj - s0, S_PAD)
                    v0 = fx_v[b, j, pl.ds(0, 16)]
                    v1 = fx_v[b, j, pl.ds(16, 16)]
                    v2 = fx_v[b, j, pl.ds(32, 16)]
                    v3 = fx_v[b, j, pl.ds(48, 16)]
                    a0 = jnp.where(neq, v0, jnp.maximum(a0, v0))
                    a1 = jnp.where(neq, v1, jnp.maximum(a1, v1))
                    a2 = jnp.where(neq, v2, jnp.maximum(a2, v2))
                    a3 = jnp.where(neq, v3, jnp.maximum(a3, v3))
                    stage_v[slot, pl.ds(0, 32)] = plsc.pack(
                        a0, a1, format=plsc.PackFormat.INTERLEAVED)
                    stage_v[slot, pl.ds(32, 32)] = plsc.pack(
                        a2, a3, format=plsc.PackFormat.INTERLEAVED)
                    prev_id = idj
                return (prev_id, a0, a1, a2, a3)

            carry = lax.fori_loop(0, T1 // 16, grp1, carry)

            @pl.when(jnp.logical_and(t - 1 >= 0, t - 1 < ntiles))
            def _(b=b):
                wrl_wait((b + 2) % 3)

            @pl.when(t + 2 < nsteps)
            def _(t=t, b=b):
                ld_start(t + 2, (b + 2) % 3)

        return carry

    z = jnp.zeros((16,), jnp.float32)
    lax.fori_loop(0, nsteps // 3, pair1, (jnp.int32(-1), z, z, z, z))

    @pl.when(jnp.logical_and(nsteps > 0, nsteps - 1 < ntiles))
    def _():
        wrl_wait(2)

    def ld2_start(t, b):
        base = tb0 + t * T1
        pltpu.async_copy(ids_hbm.at[pl.ds(base, T1)], ids_v.at[b],
                         ld_sems.at[b])

    def ld2_wait(b):
        pltpu.make_async_copy(ids_hbm.at[pl.ds(0, T1)], ids_v.at[b],
                              ld_sems.at[b]).wait()

    def wrr_wait(b):
        pltpu.make_async_copy(fx_v.at[b],
                              res_hbm.at[pl.ds(0, T1), pl.ds(D, D)],
                              wr_sems.at[b]).wait()

    def jlo_of(t):
        return jnp.maximum(r0 - (tb0 + t * T1), 0)

    def jhi_of(t):
        return jnp.minimum(r1 - (tb0 + t * T1), T1)

    def full_of(t):
        return jnp.logical_and(jlo_of(t) == 0, jhi_of(t) == T1)

    @pl.when(nsteps > 0)
    def _():
        ld2_start(0, 0)
        ld2_start(1, 1)

    def pair2(k, _):
        for b in range(3):
            t = 3 * k + b
            base = tb0 + t * T1
            ld2_wait(b)

            @pl.when(jnp.logical_and(t - 3 >= 0, full_of(t - 3)))
            def _(b=b):
                wrr_wait(b)

            jlo = jlo_of(t)
            jhi = jhi_of(t)

            def grp2(g, _c):
                gb = g * 16
                idvec = ids_v[b, pl.ds(gb, 16)]
                for jj in range(16):
                    j = gb + jj
                    slot = jnp.clip(idvec[jj] - s0, 0, S_PAD)
                    p01 = stage_v[slot, pl.ds(0, 32)]
                    p23 = stage_v[slot, pl.ds(32, 32)]
                    u0, u1 = plsc.unpack(p01,
                                         format=plsc.PackFormat.INTERLEAVED)
                    u2, u3 = plsc.unpack(p23,
                                         format=plsc.PackFormat.INTERLEAVED)
                    fx_v[b, j, pl.ds(0, 16)] = u0
                    fx_v[b, j, pl.ds(16, 16)] = u1
                    fx_v[b, j, pl.ds(32, 16)] = u2
                    fx_v[b, j, pl.ds(48, 16)] = u3
                return 0

            lax.fori_loop(jlo // 16, (jhi + 15) // 16, grp2, 0)

            full = jnp.logical_and(jlo == 0, jhi == T1)

            @pl.when(full)
            def _(b=b, base=base):
                pltpu.async_copy(fx_v.at[b],
                                 res_hbm.at[pl.ds(base, T1), pl.ds(D, D)],
                                 wr_sems.at[b])

            @pl.when(jnp.logical_and(jnp.logical_not(full), jhi > jlo))
            def _(b=b, base=base, jlo=jlo, jhi=jhi):
                rem = jhi - jlo
                off = jlo
                for sz in (128, 64, 32, 16, 8, 4, 2, 1):
                    cond = (rem & sz) != 0

                    @pl.when(cond)
                    def _(b=b, off=off, sz=sz):
                        pltpu.sync_copy(
                            fx_v.at[b, pl.ds(off, sz)],
                            res_hbm.at[pl.ds(base + off, sz), pl.ds(D, D)])

                    off = off + jnp.where(cond, sz, 0)

            @pl.when(t + 2 < nsteps)
            def _(t=t, b=b):
                ld2_start(t + 2, (b + 2) % 3)

        return 0

    lax.fori_loop(0, nsteps // 3, pair2, 0)

    for bb in range(3):
        tt = nsteps - 3 + bb

        @pl.when(jnp.logical_and(tt >= 0, full_of(tt)))
        def _(bb=bb):
            wrr_wait(bb)


def _segbc_sc(fX, ids_pad, bounds_tbl):
    mesh = plsc.VectorSubcoreMesh(core_axis_name="c", subcore_axis_name="s")
    return pl.kernel(
        _segbc_body,
        out_type=jax.ShapeDtypeStruct((N, 2 * D), jnp.float32),
        mesh=mesh,
        compiler_params=pltpu.CompilerParams(use_tc_tiling_on_sc=False,
                                             needs_layout_passes=False),
        scratch_types=[
            pltpu.VMEM((S_PAD + 1, D), jnp.bfloat16),
            pltpu.VMEM((3, T1, D), jnp.float32),
            pltpu.VMEM((3, T1), jnp.int32),
            pltpu.VMEM((16,), jnp.int32),
            pltpu.SemaphoreType.DMA((3,)),
            pltpu.SemaphoreType.DMA((3,)),
        ],
    )(fX, ids_pad, bounds_tbl)


def kernel(X, W, b, gamma, beta, i):
    ids = i.astype(jnp.int32)
    X2 = X.reshape(N // 2, 2 * D)
    fX = _fx_tc(X2, W, b.reshape(1, D), gamma.reshape(1, D),
                beta.reshape(1, D)).reshape(N + BLK, D)
    seg_bounds = jnp.arange(NW + 1, dtype=jnp.int32) * S_PAD
    row_bounds = jnp.searchsorted(ids, seg_bounds, side="left").astype(jnp.int32)
    bounds_tbl = jnp.concatenate(
        [row_bounds[:-1, None], row_bounds[1:, None],
         jnp.zeros((NW, 14), jnp.int32)], axis=1)
    ids_pad = jnp.concatenate([ids, jnp.zeros((3 * T1 + 16,), jnp.int32)])
    return _segbc_sc(fX, ids_pad, bounds_tbl)
